# Initial kernel scaffold; baseline (speedup 1.0000x reference)
import jax
import jax.numpy as jnp
from jax.experimental import pallas as pl


def kernel(x, edge_index, W_in, b_in, convW, convB, lnG, lnB, W_out, b_out):
    raise NotImplementedError("write your pallas kernel here")


# speedup vs baseline: 3.3193x; 3.3193x over previous
import functools

import jax
import jax.numpy as jnp
from jax import lax
from jax.experimental import pallas as pl
from jax.experimental.pallas import tpu as pltpu
from jax.experimental.pallas import tpu_sc as plsc

N = 50000
E = 800000
D_IN = 128
H = 64
HF = H // 2
NLAYERS = 3
EPS = 1e-5

NC = 2
NS = 16
NPAD = 50048
ROWS_PER_SUB = NPAD // NS
EDGES_PER_SUB = E // NS
CHUNK = 800
NFULL = EDGES_PER_SUB // CHUNK
TAIL = EDGES_PER_SUB - NFULL * CHUNK

_MESH = plsc.VectorSubcoreMesh(core_axis_name="c", subcore_axis_name="s")

BS = 1000
GRID = (N // BS,)


@functools.partial(
    pl.kernel,
    out_type=jax.ShapeDtypeStruct((NC, NPAD, HF), jnp.float32),
    mesh=_MESH,
    compiler_params=pltpu.CompilerParams(use_tc_tiling_on_sc=False,
                                         internal_scratch_in_bytes=0),
    scratch_types=[
        pltpu.VMEM((CHUNK,), jnp.int32),
        pltpu.VMEM((CHUNK,), jnp.int32),
        pltpu.VMEM((CHUNK, HF), jnp.float32),
        pltpu.VMEM_SHARED((NPAD, HF), jnp.float32),
        pltpu.SemaphoreType.DMA,
    ],
)
def _seg_sum(g_hbm, src_hbm, dst_hbm, zeros_hbm, out_hbm,
             src_v, dst_v, rows_v, accum, sem):
    c = lax.axis_index("c")
    s = lax.axis_index("s")
    row0 = s * ROWS_PER_SUB
    pltpu.sync_copy(zeros_hbm, accum.at[pl.ds(row0, ROWS_PER_SUB)])
    plsc.subcore_barrier()
    base = pl.multiple_of(s * EDGES_PER_SUB, 8)
    tab = g_hbm.at[c]

    def edge_block(off, cnt):
        sv = src_v.at[pl.ds(0, cnt)]
        dv = dst_v.at[pl.ds(0, cnt)]
        rv = rows_v.at[pl.ds(0, cnt)]
        pltpu.sync_copy(src_hbm.at[pl.ds(off, cnt)], sv)
        pltpu.sync_copy(dst_hbm.at[pl.ds(off, cnt)], dv)
        pltpu.async_copy(tab.at[sv], rv, sem).wait()
        pltpu.sync_copy(rv, accum.at[dv], add=True)

    @pl.loop(0, NFULL)
    def _(k):
        edge_block(base + k * CHUNK, CHUNK)

    edge_block(base + NFULL * CHUNK, TAIL)

    plsc.subcore_barrier()
    pltpu.sync_copy(accum.at[pl.ds(row0, ROWS_PER_SUB)],
                    out_hbm.at[c].at[pl.ds(row0, ROWS_PER_SUB)])


def _prologue_body(x_ref, w_ref, b_ref, dp_ref, h_ref, dinv_ref):
    h = jnp.dot(x_ref[...], w_ref[...], preferred_element_type=jnp.float32)
    h_ref[...] = jnp.maximum(h + b_ref[...], 0.0)
    deg = dp_ref[0, :, 0:1] + 1.0
    dinv_ref[...] = lax.rsqrt(deg)


def _prologue(x, w_in, b_in, degc):
    return pl.pallas_call(
        _prologue_body,
        grid=GRID,
        in_specs=[
            pl.BlockSpec((BS, D_IN), lambda i: (i, 0)),
            pl.BlockSpec((D_IN, H), lambda i: (0, 0)),
            pl.BlockSpec((H,), lambda i: (0,)),
            pl.BlockSpec((1, BS, HF), lambda i: (0, i, 0)),
        ],
        out_specs=[
            pl.BlockSpec((BS, H), lambda i: (i, 0)),
            pl.BlockSpec((BS, 1), lambda i: (i, 0)),
        ],
        out_shape=[
            jax.ShapeDtypeStruct((N, H), jnp.float32),
            jax.ShapeDtypeStruct((N, 1), jnp.float32),
        ],
    )(x, w_in, b_in, degc)


def _gmm_body(h_ref, w_ref, dinv_ref, g_ref):
    g = jnp.dot(h_ref[...], w_ref[...], preferred_element_type=jnp.float32)
    g = g * dinv_ref[...]
    g_ref[0] = g[:, :HF]
    g_ref[1] = g[:, HF:]


def _gmm(h, w, dinv):
    return pl.pallas_call(
        _gmm_body,
        grid=GRID,
        in_specs=[
            pl.BlockSpec((BS, H), lambda i: (i, 0)),
            pl.BlockSpec((H, H), lambda i: (0, 0)),
            pl.BlockSpec((BS, 1), lambda i: (i, 0)),
        ],
        out_specs=pl.BlockSpec((NC, BS, HF), lambda i: (0, i, 0)),
        out_shape=jax.ShapeDtypeStruct((NC, N, HF), jnp.float32),
    )(h, w, dinv)


def _norm_body(agg_ref, g_ref, dinv_ref, h_ref, gam_ref, bet_ref, cb_ref, o_ref):
    full = jnp.concatenate(
        [agg_ref[0] + g_ref[0], agg_ref[1] + g_ref[1]], axis=1)
    pre = dinv_ref[...] * full + cb_ref[...]
    mu = jnp.mean(pre, axis=1, keepdims=True)
    var = jnp.mean((pre - mu) ** 2, axis=1, keepdims=True)
    y = (pre - mu) * lax.rsqrt(var + EPS) * gam_ref[...] + bet_ref[...]
    o_ref[...] = jnp.maximum(y, 0.0) + h_ref[...]


def _norm(agg, g, dinv, h, gam, bet, cb):
    return pl.pallas_call(
        _norm_body,
        grid=GRID,
        in_specs=[
            pl.BlockSpec((NC, BS, HF), lambda i: (0, i, 0)),
            pl.BlockSpec((NC, BS, HF), lambda i: (0, i, 0)),
            pl.BlockSpec((BS, 1), lambda i: (i, 0)),
            pl.BlockSpec((BS, H), lambda i: (i, 0)),
            pl.BlockSpec((H,), lambda i: (0,)),
            pl.BlockSpec((H,), lambda i: (0,)),
            pl.BlockSpec((H,), lambda i: (0,)),
        ],
        out_specs=pl.BlockSpec((BS, H), lambda i: (i, 0)),
        out_shape=jax.ShapeDtypeStruct((N, H), jnp.float32),
    )(agg, g, dinv, h, gam, bet, cb)


def _out_body(h_ref, w_ref, b_ref, o_ref):
    o_ref[...] = jnp.sum(h_ref[...] * w_ref[...], axis=1, keepdims=True) + b_ref[0]


def _epilogue(h, w_row, b_out):
    return pl.pallas_call(
        _out_body,
        grid=GRID,
        in_specs=[
            pl.BlockSpec((BS, H), lambda i: (i, 0)),
            pl.BlockSpec((1, H), lambda i: (0, 0)),
            pl.BlockSpec((1,), lambda i: (0,)),
        ],
        out_specs=pl.BlockSpec((BS, 1), lambda i: (i, 0)),
        out_shape=jax.ShapeDtypeStruct((N, 1), jnp.float32),
    )(h, w_row, b_out)


def kernel(x, edge_index, W_in, b_in, convW, convB, lnG, lnB, W_out, b_out):
    src = edge_index[0]
    dst = edge_index[1]
    zeros32 = jnp.zeros((ROWS_PER_SUB, HF), jnp.float32)
    ones_tab = jnp.ones((NC, N, HF), jnp.float32)
    zidx = jnp.zeros((E,), jnp.int32)

    degc = _seg_sum(ones_tab, zidx, dst, zeros32)
    h, dinv = _prologue(x, W_in, b_in, degc)
    for l in range(NLAYERS):
        g = _gmm(h, convW[l], dinv)
        agg = _seg_sum(g, src, dst, zeros32)
        h = _norm(agg, g, dinv, h, lnG[l], lnB[l], convB[l])
    return _epilogue(h, W_out.reshape(1, H), b_out).reshape(-1)
